# Initial kernel scaffold; baseline (speedup 1.0000x reference)
import jax
import jax.numpy as jnp
from jax.experimental import pallas as pl


def kernel(z_e_x, codebook):
    raise NotImplementedError("write your pallas kernel here")


# speedup vs baseline: 1.2659x; 1.2659x over previous
import jax
import jax.numpy as jnp
from jax.experimental import pallas as pl

_BM = 256


def _vq_kernel(x_ref, cbt_ref, idx_ref):
    x = x_ref[...]
    cbt = cbt_ref[...]
    c_sq = jnp.sum(cbt * cbt, axis=0, keepdims=True)
    x_sq = jnp.sum(x * x, axis=1, keepdims=True)
    dots = jax.lax.dot_general(
        x, cbt, (((1,), (0,)), ((), ())),
        preferred_element_type=jnp.float32)
    s = (c_sq + x_sq) - 2.0 * dots
    m = jnp.min(s, axis=1, keepdims=True)
    iota = jax.lax.broadcasted_iota(jnp.int32, s.shape, 1)
    sentinel = jnp.int32(s.shape[1])
    idx_ref[0, 0, :] = jnp.min(jnp.where(s <= m, iota, sentinel), axis=1)


def kernel(z_e_x, codebook):
    n, d, h, w = z_e_x.shape
    m = n * h * w
    k = codebook.shape[0]
    x = jnp.transpose(z_e_x, (0, 2, 3, 1)).reshape(m, d)
    cbt = codebook.T
    nb = m // _BM
    out = pl.pallas_call(
        _vq_kernel,
        grid=(nb,),
        in_specs=[
            pl.BlockSpec((_BM, d), lambda i: (i, 0)),
            pl.BlockSpec((d, k), lambda i: (0, 0)),
        ],
        out_specs=pl.BlockSpec((1, 1, _BM), lambda i: (i, 0, 0)),
        out_shape=jax.ShapeDtypeStruct((nb, 1, _BM), jnp.int32),
    )(x, cbt)
    return out.reshape(n, h, w)
